# baseline (speedup 1.0000x reference)
import jax
import jax.numpy as jnp
from jax.experimental import pallas as pl
from jax.experimental.pallas import tpu as pltpu

VOCAB = 1000
B0 = 64
NBUF = 4


def _onehot_pipeline(x_ref, o_ref, vbuf, sems):
    n_steps = o_ref.shape[0] // B0
    iota = jax.lax.broadcasted_iota(jnp.int32, (B0, x_ref.shape[1], VOCAB), 2)

    def copy(i, slot):
        return pltpu.make_async_copy(
            vbuf.at[slot],
            o_ref.at[pl.ds(i * B0, B0)],
            sems.at[slot],
        )

    def body(i, _):
        slot = jax.lax.rem(i, NBUF)

        @pl.when(i >= NBUF)
        def _():
            copy(i - NBUF, slot).wait()

        xb = x_ref[pl.ds(i * B0, B0), :]
        vbuf[slot] = (xb[:, :, None] == iota).astype(jnp.float32)
        copy(i, slot).start()
        return ()

    jax.lax.fori_loop(0, n_steps, body, ())

    def drain(i, _):
        slot = jax.lax.rem(i, NBUF)
        copy(i, slot).wait()
        return ()

    jax.lax.fori_loop(n_steps - NBUF, n_steps, drain, ())


def kernel(x):
    n0, n1 = x.shape
    return pl.pallas_call(
        _onehot_pipeline,
        in_specs=[pl.BlockSpec(memory_space=pltpu.VMEM)],
        out_specs=pl.BlockSpec(memory_space=pl.ANY),
        out_shape=jax.ShapeDtypeStruct((n0, n1, VOCAB), jnp.float32),
        scratch_shapes=[
            pltpu.VMEM((NBUF, B0, n1, VOCAB), jnp.float32),
            pltpu.SemaphoreType.DMA((NBUF,)),
        ],
        compiler_params=pltpu.CompilerParams(
            vmem_limit_bytes=100 * 1024 * 1024,
        ),
    )(x)


# speedup vs baseline: 4.4729x; 4.4729x over previous
import jax
import jax.numpy as jnp
from jax.experimental import pallas as pl
from jax.experimental.pallas import tpu as pltpu

VOCAB = 1000
BV = 40


def _onehot_t_block(xt_ref, o_ref):
    i = pl.program_id(0)
    xt = xt_ref[...]
    v_idx = jax.lax.broadcasted_iota(
        jnp.int32, (xt_ref.shape[0], BV, xt_ref.shape[1]), 1
    ) + i * BV
    o_ref[...] = (xt[:, None, :] == v_idx).astype(jnp.float32)


def kernel(x):
    n0, n1 = x.shape
    xt = x.T
    out_t = pl.pallas_call(
        _onehot_t_block,
        grid=(VOCAB // BV,),
        in_specs=[pl.BlockSpec((n1, n0), lambda i: (0, 0))],
        out_specs=pl.BlockSpec((n1, BV, n0), lambda i: (0, i, 0)),
        out_shape=jax.ShapeDtypeStruct((n1, VOCAB, n0), jnp.float32),
        compiler_params=pltpu.CompilerParams(
            dimension_semantics=("parallel",),
            vmem_limit_bytes=100 * 1024 * 1024,
        ),
    )(xt)
    return out_t.transpose(2, 0, 1)
